# Initial kernel scaffold; baseline (speedup 1.0000x reference)
import jax
import jax.numpy as jnp
from jax.experimental import pallas as pl


def kernel(x_embed, y_embed):
    raise NotImplementedError("write your pallas kernel here")


# speedup vs baseline: 47.8900x; 47.8900x over previous
import jax
import jax.numpy as jnp
from jax.experimental import pallas as pl
from jax.experimental.pallas import tpu as pltpu

_MARGIN = 0.5
_EPS = 1e-6
_BIG = 1e6


def _loss_kernel(x_ref, y_ref, lxy_ref, lyx_ref):
    x = x_ref[:]
    y = y_ref[:]
    B = x.shape[0]
    D = x.shape[1]

    dims = (((1,), (1,)), ((), ()))
    g = jax.lax.dot_general(
        x, y, dims,
        preferred_element_type=jnp.float32,
        precision=jax.lax.Precision.HIGHEST,
    )

    ones_row = jnp.ones((1, D), dtype=jnp.float32)
    x2 = jnp.sum(x * x, axis=1, keepdims=True)
    sx = jnp.sum(x, axis=1, keepdims=True)
    y2r = jax.lax.dot_general(ones_row, y * y, dims,
                              preferred_element_type=jnp.float32,
                              precision=jax.lax.Precision.HIGHEST)
    syr = jax.lax.dot_general(ones_row, y, dims,
                              preferred_element_type=jnp.float32,
                              precision=jax.lax.Precision.HIGHEST)

    d2 = x2 + y2r - 2.0 * g + (2.0 * _EPS) * (sx - syr) + (D * _EPS * _EPS)
    dist = jnp.sqrt(jnp.maximum(d2, 0.0))

    ii = jax.lax.broadcasted_iota(jnp.int32, (B, B), 0)
    jj = jax.lax.broadcasted_iota(jnp.int32, (B, B), 1)
    eye = ii == jj
    diag_only = jnp.where(eye, dist, 0.0)
    d_col = jnp.sum(diag_only, axis=1, keepdims=True)
    d_row = jnp.sum(diag_only, axis=0, keepdims=True)
    dneg = jnp.where(eye, _BIG, dist)

    inv = 1.0 / (B * B)
    lxy = jnp.maximum(_MARGIN + d_col - dneg, 0.0)
    lyx = jnp.maximum(_MARGIN + d_row - dneg, 0.0)
    lxy_rows = jnp.sum(lxy, axis=1, keepdims=True)
    lyx_rows = jnp.sum(lyx, axis=1, keepdims=True)
    lxy_ref[:, :] = jnp.sum(lxy_rows, axis=0, keepdims=True) * inv
    lyx_ref[:, :] = jnp.sum(lyx_rows, axis=0, keepdims=True) * inv


def kernel(x_embed, y_embed):
    out_xy, out_yx = pl.pallas_call(
        _loss_kernel,
        out_shape=(
            jax.ShapeDtypeStruct((1, 1), jnp.float32),
            jax.ShapeDtypeStruct((1, 1), jnp.float32),
        ),
    )(x_embed, y_embed)
    return (out_xy[0, 0], out_yx[0, 0])


# speedup vs baseline: 67.4708x; 1.4089x over previous
import jax
import jax.numpy as jnp
from jax.experimental import pallas as pl
from jax.experimental.pallas import tpu as pltpu

_MARGIN = 0.5
_EPS = 1e-6
_HIGHEST = jax.lax.Precision.HIGHEST
_DIMS = (((1,), (1,)), ((), ()))


def _loss_kernel(x_ref, y_ref, lxy_ref, lyx_ref):
    x = x_ref[:]
    y = y_ref[:]
    B = x.shape[0]
    D = x.shape[1]

    z = x - y + _EPS
    zz = z * z
    ones_row = jnp.ones((1, D), dtype=jnp.float32)
    d_col = jnp.sqrt(jnp.sum(zz, axis=1, keepdims=True))
    d_row = jnp.sqrt(jax.lax.dot_general(
        ones_row, zz, _DIMS,
        preferred_element_type=jnp.float32, precision=_HIGHEST))

    a = jnp.sum(x * (x + 2.0 * _EPS), axis=1, keepdims=True)
    b = jnp.sum(y * (y - 2.0 * _EPS), axis=1, keepdims=True) + D * _EPS * _EPS
    ones_col = jnp.ones((B, 1), dtype=jnp.float32)
    x_aug = jnp.concatenate([x, a, ones_col], axis=1)
    y_aug = jnp.concatenate([-2.0 * y, ones_col, b], axis=1)
    d2 = jax.lax.dot_general(
        x_aug, y_aug, _DIMS,
        preferred_element_type=jnp.float32, precision=_HIGHEST)
    d2c = jnp.maximum(d2, 0.0)
    dist = d2c * jax.lax.rsqrt(d2c + 1e-30)

    inv = 1.0 / (B * B)
    diag_corr = B * _MARGIN
    c_col = _MARGIN + d_col
    c_row = _MARGIN + d_row
    lxy = jnp.maximum(c_col - dist, 0.0)
    lyx = jnp.maximum(c_row - dist, 0.0)
    lxy_rows = jnp.sum(lxy, axis=1, keepdims=True)
    lyx_rows = jnp.sum(lyx, axis=1, keepdims=True)
    lxy_ref[:, :] = (jnp.sum(lxy_rows, axis=0, keepdims=True) - diag_corr) * inv
    lyx_ref[:, :] = (jnp.sum(lyx_rows, axis=0, keepdims=True) - diag_corr) * inv


def kernel(x_embed, y_embed):
    out_xy, out_yx = pl.pallas_call(
        _loss_kernel,
        out_shape=(
            jax.ShapeDtypeStruct((1, 1), jnp.float32),
            jax.ShapeDtypeStruct((1, 1), jnp.float32),
        ),
    )(x_embed, y_embed)
    return (out_xy[0, 0], out_yx[0, 0])


# speedup vs baseline: 104.4279x; 1.5478x over previous
import jax
import jax.numpy as jnp
from jax.experimental import pallas as pl
from jax.experimental.pallas import tpu as pltpu

_MARGIN = 0.5
_EPS = 1e-6
_DIMS = (((1,), (1,)), ((), ()))


def _loss_kernel(x_ref, y_ref, lxy_ref, lyx_ref):
    x = x_ref[:]
    y = y_ref[:]
    B = x.shape[0]
    D = x.shape[1]

    z = x - y + _EPS
    zz = z * z
    ones_row = jnp.ones((1, D), dtype=jnp.float32)
    d_col = jnp.sqrt(jnp.sum(zz, axis=1, keepdims=True))
    d_row = jnp.sqrt(jax.lax.dot_general(
        ones_row, zz, _DIMS,
        preferred_element_type=jnp.float32))

    a = jnp.sum(x * (x + 2.0 * _EPS), axis=1, keepdims=True)
    b = jnp.sum(y * (y - 2.0 * _EPS), axis=1, keepdims=True) + D * _EPS * _EPS
    ones_col = jnp.ones((B, 1), dtype=jnp.float32)
    x_aug = jnp.concatenate([x, a, ones_col], axis=1)
    y_aug = jnp.concatenate([-2.0 * y, ones_col, b], axis=1)
    d2 = jax.lax.dot_general(
        x_aug, y_aug, _DIMS,
        preferred_element_type=jnp.float32)
    d2c = jnp.maximum(d2, 0.0)
    dist = d2c * jax.lax.rsqrt(d2c + 1e-30)

    inv = 1.0 / (B * B)
    diag_corr = B * _MARGIN
    c_col = _MARGIN + d_col
    c_row = _MARGIN + d_row
    lxy = jnp.maximum(c_col - dist, 0.0)
    lyx = jnp.maximum(c_row - dist, 0.0)
    lxy_rows = jnp.sum(lxy, axis=1, keepdims=True)
    lyx_rows = jnp.sum(lyx, axis=1, keepdims=True)
    lxy_ref[:, :] = (jnp.sum(lxy_rows, axis=0, keepdims=True) - diag_corr) * inv
    lyx_ref[:, :] = (jnp.sum(lyx_rows, axis=0, keepdims=True) - diag_corr) * inv


def kernel(x_embed, y_embed):
    out_xy, out_yx = pl.pallas_call(
        _loss_kernel,
        out_shape=(
            jax.ShapeDtypeStruct((1, 1), jnp.float32),
            jax.ShapeDtypeStruct((1, 1), jnp.float32),
        ),
    )(x_embed, y_embed)
    return (out_xy[0, 0], out_yx[0, 0])


# speedup vs baseline: 105.2236x; 1.0076x over previous
import jax
import jax.numpy as jnp
from jax.experimental import pallas as pl
from jax.experimental.pallas import tpu as pltpu

_MARGIN = 0.5
_EPS = 1e-6
_DIMS = (((1,), (1,)), ((), ()))


def _loss_kernel(x_ref, y_ref, lxy_ref, lyx_ref):
    x = x_ref[:]
    y = y_ref[:]
    B = x.shape[0]
    D = x.shape[1]

    z = x - y + _EPS
    zz = z * z
    ones_row = jnp.ones((1, D), dtype=jnp.float32)
    d_col = jnp.sqrt(jnp.sum(zz, axis=1, keepdims=True))
    d_row = jnp.sqrt(jax.lax.dot_general(
        ones_row, zz, _DIMS,
        preferred_element_type=jnp.float32))

    a = jnp.sum(x * (x + 2.0 * _EPS), axis=1, keepdims=True)
    b = jnp.sum(y * (y - 2.0 * _EPS), axis=1, keepdims=True) + D * _EPS * _EPS
    ones_col = jnp.ones((B, 1), dtype=jnp.float32)
    x_aug = jnp.concatenate([x, a, ones_col], axis=1)
    y_aug = jnp.concatenate([-2.0 * y, ones_col, b], axis=1)
    d2 = jax.lax.dot_general(
        x_aug, y_aug, _DIMS,
        preferred_element_type=jnp.float32)
    inv = 1.0 / (B * B)
    diag_corr = B * _MARGIN
    c_col = _MARGIN + d_col
    c_row = _MARGIN + d_row

    chunk = 256
    sxy = jnp.zeros((1, B), dtype=jnp.float32)
    syx = jnp.zeros((1, B), dtype=jnp.float32)
    for k in range(0, B, chunk):
        m = jnp.maximum(d2[k:k + chunk, :], 1e-30)
        dist = m * jax.lax.rsqrt(m)
        lxy = jnp.maximum(c_col[k:k + chunk, :] - dist, 0.0)
        lyx = jnp.maximum(c_row - dist, 0.0)
        sxy = sxy + jnp.sum(lxy, axis=0, keepdims=True)
        syx = syx + jnp.sum(lyx, axis=0, keepdims=True)
    lxy_ref[:, :] = (jnp.sum(sxy, axis=1, keepdims=True) - diag_corr) * inv
    lyx_ref[:, :] = (jnp.sum(syx, axis=1, keepdims=True) - diag_corr) * inv


def kernel(x_embed, y_embed):
    out_xy, out_yx = pl.pallas_call(
        _loss_kernel,
        out_shape=(
            jax.ShapeDtypeStruct((1, 1), jnp.float32),
            jax.ShapeDtypeStruct((1, 1), jnp.float32),
        ),
    )(x_embed, y_embed)
    return (out_xy[0, 0], out_yx[0, 0])


# speedup vs baseline: 118.2890x; 1.1242x over previous
import jax
import jax.numpy as jnp
from jax.experimental import pallas as pl
from jax.experimental.pallas import tpu as pltpu

_MARGIN = 0.5
_EPS = 1e-6
_DIMS = (((1,), (1,)), ((), ()))


def _loss_kernel(x_ref, y_ref, lxy_ref, lyx_ref):
    x = x_ref[:]
    y = y_ref[:]
    B = x.shape[0]
    D = x.shape[1]

    z = x - y + _EPS
    zz = z * z
    ones_row = jnp.ones((1, D), dtype=jnp.float32)
    s_col = jnp.maximum(jnp.sum(zz, axis=1, keepdims=True), 1e-30)
    d_col = s_col * jax.lax.rsqrt(s_col)
    s_row = jnp.maximum(jax.lax.dot_general(
        ones_row, zz, _DIMS,
        preferred_element_type=jnp.float32), 1e-30)
    d_row = s_row * jax.lax.rsqrt(s_row)

    a = jnp.sum(x * (x + 2.0 * _EPS), axis=1, keepdims=True)
    b = jnp.sum(y * (y - 2.0 * _EPS), axis=1, keepdims=True) + D * _EPS * _EPS
    ones_col = jnp.ones((B, 1), dtype=jnp.float32)
    x_aug = jnp.concatenate([x, a, ones_col], axis=1)
    y_aug = jnp.concatenate([-2.0 * y, ones_col, b], axis=1)
    d2 = jax.lax.dot_general(
        x_aug, y_aug, _DIMS,
        preferred_element_type=jnp.float32)
    inv = 1.0 / (B * B)
    diag_corr = B * _MARGIN
    c_col = (_MARGIN + d_col).astype(jnp.bfloat16)
    c_row = (_MARGIN + d_row).astype(jnp.bfloat16)

    m = jnp.maximum(d2.astype(jnp.bfloat16), jnp.bfloat16(1e-30))
    dist = m * jax.lax.rsqrt(m)
    zero = jnp.bfloat16(0.0)
    lxy = jnp.maximum(c_col - dist, zero)
    lyx = jnp.maximum(c_row - dist, zero)
    ones_b = jnp.ones((1, B), dtype=jnp.bfloat16)
    red_dims = (((1,), (0,)), ((), ()))
    sxy = jax.lax.dot_general(ones_b, lxy, red_dims,
                              preferred_element_type=jnp.float32)
    syx = jax.lax.dot_general(ones_b, lyx, red_dims,
                              preferred_element_type=jnp.float32)
    lxy_ref[:, :] = (jnp.sum(sxy, axis=1, keepdims=True) - diag_corr) * inv
    lyx_ref[:, :] = (jnp.sum(syx, axis=1, keepdims=True) - diag_corr) * inv


def kernel(x_embed, y_embed):
    out_xy, out_yx = pl.pallas_call(
        _loss_kernel,
        out_shape=(
            jax.ShapeDtypeStruct((1, 1), jnp.float32),
            jax.ShapeDtypeStruct((1, 1), jnp.float32),
        ),
    )(x_embed, y_embed)
    return (out_xy[0, 0], out_yx[0, 0])
